# baseline (speedup 1.0000x reference)
import dataclasses

import jax
import jax.numpy as jnp
from jax import lax
from jax.experimental import pallas as pl
from jax.experimental.pallas import tpu as pltpu
from jax.experimental.pallas import tpu_sc as plsc

G = 128
D = 28
N = 1048576
V = G * G * G

NC = 2
NS = 16
NW = NC * NS
L = 16

DP = 32
W = 128
PPW = N // NW
NWIN = PPW // W

RC = 512
SKEW = RC
RPW = V // NW

CORNER_OFFS = [(dx * G + dy) * G + dz
               for dx in (0, 1) for dy in (0, 1) for dz in (0, 1)]


def _repack_body(tfm_hbm, t32_hbm, f_v, out_v, sem):
    wid = lax.axis_index("s") * NC + lax.axis_index("c")
    base = wid * RPW
    i0 = lax.iota(jnp.int32, 16) * SKEW
    i1 = i0 + (D - L) * SKEW

    @pl.loop(0, RPW, step=RC)
    def _chunk(r0):
        v0 = base + r0
        copies = [pltpu.make_async_copy(tfm_hbm.at[pl.ds(d * V + v0, RC)],
                                        f_v.at[pl.ds(d * SKEW, RC)], sem)
                  for d in range(D)]
        for d in range(D):
            copies[d].start()
        for d in range(D):
            copies[d].wait()

        @pl.loop(0, RC)
        def _row(r):
            rv = jnp.full((L,), r, jnp.int32)
            out_v[r, pl.ds(0, L)] = plsc.load_gather(f_v, [i0 + rv])
            out_v[r, pl.ds(D - L, L)] = plsc.load_gather(f_v, [i1 + rv])

        pltpu.sync_copy(out_v, t32_hbm.at[pl.ds(v0, RC)])


def _main_body(pos_hbm, table_hbm, out_hbm, pos_v, idx_v, wt_v, cor_v, out_v,
               sem):
    wid = lax.axis_index("s") * NC + lax.axis_index("c")

    @pl.loop(0, NWIN)
    def _window(win):
        base = wid * PPW + win * W
        pcopies = [pltpu.make_async_copy(pos_hbm.at[pl.ds(d * N + base, W)],
                                         pos_v.at[d], sem)
                   for d in range(3)]
        for d in range(3):
            pcopies[d].start()
        for d in range(3):
            pcopies[d].wait()

        @pl.loop(0, W, step=L)
        def _grp(g):
            xs = pos_v[0, pl.ds(g, L)] * jnp.float32(G - 1)
            ys = pos_v[1, pl.ds(g, L)] * jnp.float32(G - 1)
            zs = pos_v[2, pl.ds(g, L)] * jnp.float32(G - 1)
            x0 = jnp.minimum(jnp.maximum(xs.astype(jnp.int32), 0), G - 2)
            y0 = jnp.minimum(jnp.maximum(ys.astype(jnp.int32), 0), G - 2)
            z0 = jnp.minimum(jnp.maximum(zs.astype(jnp.int32), 0), G - 2)
            fx = xs - x0.astype(jnp.float32)
            fy = ys - y0.astype(jnp.float32)
            fz = zs - z0.astype(jnp.float32)
            gx = jnp.float32(1.0) - fx
            gy = jnp.float32(1.0) - fy
            gz = jnp.float32(1.0) - fz
            flat = (x0 * G + y0) * G + z0
            wxs = (gx, fx)
            wys = (gy, fy)
            wzs = (gz, fz)
            for c in range(8):
                dx, dy, dz = (c >> 2) & 1, (c >> 1) & 1, c & 1
                idx_v[c, pl.ds(g, L)] = flat + CORNER_OFFS[c]
                wt_v[c, pl.ds(g, L)] = wxs[dx] * wys[dy] * wzs[dz]

        copies = [pltpu.make_async_copy(table_hbm.at[idx_v.at[c]],
                                        cor_v.at[c], sem)
                  for c in range(8)]
        for c in range(8):
            copies[c].start()
        for c in range(8):
            copies[c].wait()

        @pl.loop(0, W)
        def _pt(w):
            wsp = jnp.full((L,), w, jnp.int32)
            acc0 = None
            acc1 = None
            for c in range(8):
                ws = plsc.load_gather(wt_v, [jnp.full((L,), c, jnp.int32), wsp])
                r0 = cor_v[c, w, pl.ds(0, L)]
                r1 = cor_v[c, w, pl.ds(D - L, L)]
                if acc0 is None:
                    acc0 = ws * r0
                    acc1 = ws * r1
                else:
                    acc0 = acc0 + ws * r0
                    acc1 = acc1 + ws * r1
            w28 = w * D
            out_v[pl.ds(w28, L)] = acc0
            out_v[pl.ds(w28 + D - L, L)] = acc1

        pltpu.sync_copy(out_v, out_hbm.at[pl.ds(base * D, W * D)])


def _make_cp():
    cp = pltpu.CompilerParams()
    for field, val in (("needs_layout_passes", False),
                       ("use_tc_tiling_on_sc", False)):
        if field in pltpu.CompilerParams.__dataclass_fields__:
            cp = dataclasses.replace(cp, **{field: val})
    return cp


def kernel(positions, voxel_grid):
    mesh = plsc.VectorSubcoreMesh(core_axis_name="c", subcore_axis_name="s")
    cp = _make_cp()

    repack = pl.kernel(
        _repack_body,
        out_type=jax.ShapeDtypeStruct((V, DP), jnp.float32),
        mesh=mesh,
        compiler_params=cp,
        scratch_types=[
            pltpu.VMEM((D * SKEW,), jnp.float32),
            pltpu.VMEM((RC, DP), jnp.float32),
            pltpu.SemaphoreType.DMA,
        ],
    )

    main = pl.kernel(
        _main_body,
        out_type=jax.ShapeDtypeStruct((N * D,), jnp.float32),
        mesh=mesh,
        compiler_params=cp,
        scratch_types=[
            pltpu.VMEM((3, W), jnp.float32),
            pltpu.VMEM((8, W), jnp.int32),
            pltpu.VMEM((8, W), jnp.float32),
            pltpu.VMEM((8, W, DP), jnp.float32),
            pltpu.VMEM((W * D,), jnp.float32),
            pltpu.SemaphoreType.DMA,
        ],
    )

    t32 = repack(voxel_grid.T.reshape(D * V))
    out1d = main(positions.T.reshape(3 * N), t32)
    return out1d.reshape(N, D)
